# Initial kernel scaffold; baseline (speedup 1.0000x reference)
import jax
import jax.numpy as jnp
from jax.experimental import pallas as pl


def kernel(h, edge_index, edge_x, ep_w1, ep_b1, ep_w2, ep_b2, up_w1, up_b1, up_w2, up_b2, ln_g, ln_b):
    raise NotImplementedError("write your pallas kernel here")


# speedup vs baseline: 1.4498x; 1.4498x over previous
import functools

import jax
import jax.numpy as jnp
from jax import lax
from jax.experimental import pallas as pl
from jax.experimental.pallas import tpu as pltpu
from jax.experimental.pallas import tpu_sc as plsc

_N = 10000
_E = 320000
_D = 128

_K = 40
_TILES = 16
_EPT = _E // _TILES
_CHUNKS = _EPT // _K
_NP = 10240
_RPT = _NP // _TILES
_ZR = 128

_DOT_T = (((1,), (1,)), ((), ()))


def _node_proj_body(h_ref, w1a_ref, w1b_ref, a_ref, b_ref):
    hb = h_ref[...]
    a_ref[...] = lax.dot_general(hb, w1a_ref[...], _DOT_T,
                                 preferred_element_type=jnp.float32)
    b_ref[...] = lax.dot_general(hb, w1b_ref[...], _DOT_T,
                                 preferred_element_type=jnp.float32)


def _edge_proj_body(x_ref, w1c_ref, b1_ref, c_ref):
    c_ref[...] = lax.dot_general(x_ref[...], w1c_ref[...], _DOT_T,
                                 preferred_element_type=jnp.float32) + b1_ref[...]


def _node_update_body(h_ref, g0_ref, g1_ref, d0_ref, d1_ref, w2_ref, b2_ref,
                      u1_ref, ub1_ref, u2_ref, ub2_ref, lg_ref, lb_ref, o_ref):
    hb = h_ref[...]
    w2 = w2_ref[...]
    b2 = b2_ref[...]
    in_deg = d0_ref[:, 0:1]
    out_deg = d1_ref[:, 0:1]
    incoming = (lax.dot_general(g0_ref[...], w2, _DOT_T,
                                preferred_element_type=jnp.float32)
                + in_deg * b2)
    outgoing = (lax.dot_general(g1_ref[...], w2, _DOT_T,
                                preferred_element_type=jnp.float32)
                + out_deg * b2)
    incoming = incoming / jnp.maximum(in_deg, 1.0)
    outgoing = outgoing / jnp.maximum(out_deg, 1.0)
    u1 = u1_ref[...]
    u_hid = jnp.maximum(
        lax.dot_general(hb, u1[:, :128], _DOT_T,
                        preferred_element_type=jnp.float32)
        + lax.dot_general(incoming, u1[:, 128:256], _DOT_T,
                          preferred_element_type=jnp.float32)
        + lax.dot_general(outgoing, u1[:, 256:384], _DOT_T,
                          preferred_element_type=jnp.float32)
        + ub1_ref[...], 0.0)
    updated = lax.dot_general(u_hid, u2_ref[...], _DOT_T,
                              preferred_element_type=jnp.float32) + ub2_ref[...]
    y = hb + updated
    mean = jnp.mean(y, axis=-1, keepdims=True)
    var = jnp.mean((y - mean) ** 2, axis=-1, keepdims=True)
    o_ref[...] = (y - mean) * lax.rsqrt(var + 1e-5) * lg_ref[...] + lb_ref[...]


def _sc_edge_aggregate(a, b, c, src, dst):
    mesh = plsc.VectorSubcoreMesh(core_axis_name="c", subcore_axis_name="s",
                                  num_cores=2, num_subcores=_TILES)

    @functools.partial(
        pl.kernel,
        out_type=[jax.ShapeDtypeStruct((_NP, _D), jnp.float32),
                  jax.ShapeDtypeStruct((_NP, _D), jnp.float32),
                  jax.ShapeDtypeStruct((_NP, _D), jnp.float32),
                  jax.ShapeDtypeStruct((_NP, _D), jnp.float32)],
        mesh=mesh,
        scratch_types=[
            pltpu.VMEM((_K,), jnp.int32),
            pltpu.VMEM((_K,), jnp.int32),
            pltpu.VMEM((_K,), jnp.int32),
            pltpu.VMEM((_K, _D), jnp.float32),
            pltpu.VMEM((_K, _D), jnp.float32),
            pltpu.VMEM((_K, _D), jnp.float32),
            pltpu.VMEM((_K, _D), jnp.float32),
            pltpu.VMEM((_ZR, _D), jnp.float32),
            pltpu.VMEM((_K, _D), jnp.float32),
            pltpu.VMEM_SHARED((_NP, _D), jnp.float32),
            pltpu.SemaphoreType.DMA,
            pltpu.SemaphoreType.DMA,
            pltpu.SemaphoreType.DMA,
        ],
    )
    def sc_kernel(a_hbm, b_hbm, c_hbm, src_hbm, dst_hbm, agg_in_hbm,
                  agg_out_hbm, deg_in_hbm, deg_out_hbm, src_v, dst_v, scat_v,
                  av, bv, cv, hv, zv, ov, acc, sem_a, sem_b, sem_c):
        cid = lax.axis_index("c")
        sid = lax.axis_index("s")

        zvec = jnp.zeros((16,), jnp.float32)
        onevec = jnp.full((16,), 1.0, jnp.float32)

        def zb_body(t, _):
            zv[t // 8, pl.ds((t % 8) * 16, 16)] = zvec
            return 0
        lax.fori_loop(0, _ZR * 8, zb_body, 0)

        def ob_body(t, _):
            ov[t // 8, pl.ds((t % 8) * 16, 16)] = onevec
            return 0
        lax.fori_loop(0, _K * 8, ob_body, 0)

        row0 = sid * _RPT

        def zero_acc():
            def zacc_body(t, _):
                pltpu.sync_copy(zv, acc.at[pl.ds(row0 + t * _ZR, _ZR)])
                return 0
            lax.fori_loop(0, _RPT // _ZR, zacc_body, 0)

        zero_acc()
        plsc.subcore_barrier()

        def load_scat_idx(base):
            @pl.when(cid == 0)
            def _():
                pltpu.sync_copy(dst_hbm.at[pl.ds(base, _K)], scat_v)

            @pl.when(cid == 1)
            def _():
                pltpu.sync_copy(src_hbm.at[pl.ds(base, _K)], scat_v)

        def deg_chunk(i, _):
            base = sid * _EPT + i * _K
            load_scat_idx(base)
            pltpu.sync_copy(ov, acc.at[scat_v], add=True)
            return 0
        lax.fori_loop(0, _CHUNKS, deg_chunk, 0)

        plsc.subcore_barrier()

        @pl.when(cid == 0)
        def _():
            pltpu.sync_copy(acc.at[pl.ds(row0, _RPT)],
                            deg_in_hbm.at[pl.ds(row0, _RPT)])

        @pl.when(cid == 1)
        def _():
            pltpu.sync_copy(acc.at[pl.ds(row0, _RPT)],
                            deg_out_hbm.at[pl.ds(row0, _RPT)])

        plsc.subcore_barrier()
        zero_acc()
        plsc.subcore_barrier()

        plsc.subcore_barrier()

        def chunk_body(i, _):
            base = sid * _EPT + i * _K
            pltpu.sync_copy(src_hbm.at[pl.ds(base, _K)], src_v)
            pltpu.sync_copy(dst_hbm.at[pl.ds(base, _K)], dst_v)
            load_scat_idx(base)
            cp_a = pltpu.async_copy(a_hbm.at[src_v], av, sem_a)
            cp_b = pltpu.async_copy(b_hbm.at[dst_v], bv, sem_b)
            cp_c = pltpu.async_copy(c_hbm.at[pl.ds(base, _K)], cv, sem_c)
            cp_a.wait()
            cp_b.wait()
            cp_c.wait()

            def row_body(r, _):
                for j in range(_D // 16):
                    s = pl.ds(j * 16, 16)
                    hv[r, s] = jnp.maximum(av[r, s] + bv[r, s] + cv[r, s], 0.0)
                return 0
            lax.fori_loop(0, _K, row_body, 0)

            pltpu.sync_copy(hv, acc.at[scat_v], add=True)
            return 0
        lax.fori_loop(0, _CHUNKS, chunk_body, 0)

        plsc.subcore_barrier()

        @pl.when(cid == 0)
        def _():
            pltpu.sync_copy(acc.at[pl.ds(row0, _RPT)],
                            agg_in_hbm.at[pl.ds(row0, _RPT)])

        @pl.when(cid == 1)
        def _():
            pltpu.sync_copy(acc.at[pl.ds(row0, _RPT)],
                            agg_out_hbm.at[pl.ds(row0, _RPT)])

    return sc_kernel(a, b, c, src, dst)


def kernel(h, edge_index, edge_x, ep_w1, ep_b1, ep_w2, ep_b2,
           up_w1, up_b1, up_w2, up_b2, ln_g, ln_b):
    w1a = ep_w1[:, :_D]
    w1b = ep_w1[:, _D:2 * _D]
    w1c = ep_w1[:, 2 * _D:]
    b1 = ep_b1.reshape(1, _D)

    nblk = 1000
    a, b = pl.pallas_call(
        _node_proj_body,
        grid=(_N // nblk,),
        in_specs=[
            pl.BlockSpec((nblk, _D), lambda i: (i, 0)),
            pl.BlockSpec((_D, _D), lambda i: (0, 0)),
            pl.BlockSpec((_D, _D), lambda i: (0, 0)),
        ],
        out_specs=[
            pl.BlockSpec((nblk, _D), lambda i: (i, 0)),
            pl.BlockSpec((nblk, _D), lambda i: (i, 0)),
        ],
        out_shape=[jax.ShapeDtypeStruct((_N, _D), jnp.float32),
                   jax.ShapeDtypeStruct((_N, _D), jnp.float32)],
    )(h, w1a, w1b)

    eblk = 2560
    c = pl.pallas_call(
        _edge_proj_body,
        grid=(_E // eblk,),
        in_specs=[
            pl.BlockSpec((eblk, _D), lambda i: (i, 0)),
            pl.BlockSpec((_D, _D), lambda i: (0, 0)),
            pl.BlockSpec((1, _D), lambda i: (0, 0)),
        ],
        out_specs=pl.BlockSpec((eblk, _D), lambda i: (i, 0)),
        out_shape=jax.ShapeDtypeStruct((_E, _D), jnp.float32),
    )(edge_x, w1c, b1)

    agg_in, agg_out, deg_in, deg_out = _sc_edge_aggregate(
        a, b, c, edge_index[0], edge_index[1])

    out = pl.pallas_call(
        _node_update_body,
        grid=(_N // nblk,),
        in_specs=[
            pl.BlockSpec((nblk, _D), lambda i: (i, 0)),
            pl.BlockSpec((nblk, _D), lambda i: (i, 0)),
            pl.BlockSpec((nblk, _D), lambda i: (i, 0)),
            pl.BlockSpec((nblk, _D), lambda i: (i, 0)),
            pl.BlockSpec((nblk, _D), lambda i: (i, 0)),
            pl.BlockSpec((_D, _D), lambda i: (0, 0)),
            pl.BlockSpec((1, _D), lambda i: (0, 0)),
            pl.BlockSpec((_D, 3 * _D), lambda i: (0, 0)),
            pl.BlockSpec((1, _D), lambda i: (0, 0)),
            pl.BlockSpec((_D, _D), lambda i: (0, 0)),
            pl.BlockSpec((1, _D), lambda i: (0, 0)),
            pl.BlockSpec((1, _D), lambda i: (0, 0)),
            pl.BlockSpec((1, _D), lambda i: (0, 0)),
        ],
        out_specs=pl.BlockSpec((nblk, _D), lambda i: (i, 0)),
        out_shape=jax.ShapeDtypeStruct((_N, _D), jnp.float32),
    )(h, agg_in, agg_out, deg_in, deg_out,
      ep_w2, ep_b2.reshape(1, _D), up_w1, up_b1.reshape(1, _D), up_w2,
      up_b2.reshape(1, _D), ln_g.reshape(1, _D), ln_b.reshape(1, _D))
    return out
